# baseline (speedup 1.0000x reference)
import functools

import jax
import jax.numpy as jnp
from jax import lax
from jax.experimental import pallas as pl
from jax.experimental.pallas import tpu as pltpu
from jax.experimental.pallas import tpu_sc as plsc

N_NODES = 10000
N_EDGES = 320000
C_S = 128
C_V = 8
C_Z = 16
NS_PTS = 8
D_POINTS = 10.0

_NC = 2
_NS = 16
_NW = _NC * _NS
_EPW = N_EDGES // _NW
_CH = 40
_NCHUNK = _EPW // _CH
_N_PAD = 10240
_ROWS_PER_TILE = _N_PAD // _NS


def _node_kernel(ns_ref, rot_ref, tr_ref, nvt_ref, wpts_ref, wcomb_ref,
                 w1_ref, w2_ref, w46_ref, w56_ref, b_ref,
                 vcat_ref, a_ref, b_out_ref):
    ns = ns_ref[...]
    rot = rot_ref[...]
    tr = tr_ref[...]
    nvt = nvt_ref[...]
    wpts = wpts_ref[...]
    wcomb = wcomb_ref[...]
    wc_top = wcomb[:C_V, :]
    wc_bot = wcomb[C_V:, :]
    p = jnp.dot(ns, wpts, preferred_element_type=jnp.float32)
    a_acc = jnp.dot(ns, w1_ref[...], preferred_element_type=jnp.float32)
    a_acc = a_acc + b_ref[...]
    b_acc = jnp.dot(ns, w2_ref[...], preferred_element_type=jnp.float32)
    for i in range(3):
        r_i = (rot[:, 3 * i + 0:3 * i + 1] * p[:, 0:NS_PTS]
               + rot[:, 3 * i + 1:3 * i + 2] * p[:, NS_PTS:2 * NS_PTS]
               + rot[:, 3 * i + 2:3 * i + 3] * p[:, 2 * NS_PTS:3 * NS_PTS])
        nv_i = nvt[:, C_V * i:C_V * (i + 1)]
        out_v_i = (jnp.dot(nv_i, wc_top, preferred_element_type=jnp.float32)
                   + jnp.dot(r_i, wc_bot, preferred_element_type=jnp.float32)
                   + tr[:, i:i + 1] * (1.0 / D_POINTS))
        vcat_ref[:, C_V * i:C_V * (i + 1)] = out_v_i
        a_acc = a_acc + jnp.dot(out_v_i, w46_ref[C_V * i:C_V * (i + 1), :],
                                preferred_element_type=jnp.float32)
        b_acc = b_acc + jnp.dot(out_v_i, w56_ref[C_V * i:C_V * (i + 1), :],
                                preferred_element_type=jnp.float32)
    a_ref[...] = a_acc
    b_out_ref[...] = b_acc


def _node_precompute(ns, rot9, trans, nvt, W_pts, W_comb, W1, W2, W46, W56, b):
    rows = 1000
    grid = N_NODES // rows
    full = lambda shape: pl.BlockSpec(shape, lambda i: (0, 0))
    blk = lambda w: pl.BlockSpec((rows, w), lambda i: (i, 0))
    return pl.pallas_call(
        _node_kernel,
        grid=(grid,),
        in_specs=[
            blk(C_S), blk(9), blk(3), blk(3 * C_V),
            full((C_S, 3 * NS_PTS)), full((C_V + NS_PTS, C_V)),
            full((C_S, C_S)), full((C_S, C_S)),
            full((3 * C_V, C_S)), full((3 * C_V, C_S)), full((1, C_S)),
        ],
        out_specs=[blk(3 * C_V), blk(C_S), blk(C_S)],
        out_shape=[
            jax.ShapeDtypeStruct((N_NODES, 3 * C_V), jnp.float32),
            jax.ShapeDtypeStruct((N_NODES, C_S), jnp.float32),
            jax.ShapeDtypeStruct((N_NODES, C_S), jnp.float32),
        ],
    )(ns, rot9, trans, nvt, W_pts, W_comb, W1, W2, W46, W56, b)


def _edgec_kernel(ef_ref, w3_ref, c_ref):
    c_ref[...] = jnp.dot(ef_ref[...], w3_ref[...],
                         preferred_element_type=jnp.float32)


def _edge_c(ef, W3):
    rows = 8000
    grid = N_EDGES // rows
    return pl.pallas_call(
        _edgec_kernel,
        grid=(grid,),
        in_specs=[
            pl.BlockSpec((rows, C_Z), lambda i: (i, 0)),
            pl.BlockSpec((C_Z, C_S), lambda i: (0, 0)),
        ],
        out_specs=pl.BlockSpec((rows, C_S), lambda i: (i, 0)),
        out_shape=jax.ShapeDtypeStruct((N_EDGES, C_S), jnp.float32),
    )(ef, W3)


def _softplus_vec(x):
    t = jnp.exp(jnp.minimum(x, -x))
    p = t * (0.9721037 - 0.2992710 * t)
    return jnp.maximum(x, 0.0) + p


def _sc_edge_body(a_hbm, b_hbm, c_hbm, src_hbm, dst_hbm, z_hbm, out_hbm,
                  si0, di0, sd0, si1, di1, sd1,
                  a0, b0, c0, r0buf, a1, b1, c1, r1buf,
                  acc, sem_g0, sem_g1, sem_s0, sem_s1, sem_i0, sem_i1):
    cid = lax.axis_index("c")
    sid = lax.axis_index("s")
    wid = cid * _NS + sid
    cb = wid * _EPW

    row0 = sid * _ROWS_PER_TILE
    pltpu.sync_copy(z_hbm.at[pl.ds(row0, _ROWS_PER_TILE)],
                    acc.at[pl.ds(row0, _ROWS_PER_TILE)])
    plsc.subcore_barrier()

    def i_descs(j, si, di, sem):
        return (pltpu.make_async_copy(src_hbm.at[wid, j], si, sem),
                pltpu.make_async_copy(dst_hbm.at[wid, j], di, sem))

    def g_descs(j, si, di, a, b, c, sem):
        return (pltpu.make_async_copy(a_hbm.at[si], a, sem),
                pltpu.make_async_copy(b_hbm.at[di], b, sem),
                pltpu.make_async_copy(c_hbm.at[pl.ds(cb + j * _CH, _CH)],
                                      c, sem))

    def s_desc(res, sdi, sem):
        return pltpu.make_async_copy(res, acc.at[sdi], sem)

    def compute(a, b, c, res):
        def row_body(r):
            for v in range(C_S // 16):
                sl = pl.ds(16 * v, 16)
                res[r, sl] = _softplus_vec(a[r, sl] + b[r, sl] + c[r, sl])
        plsc.parallel_loop(0, _CH, 1, unroll=4)(row_body)

    slots = ((si0, di0, sd0, a0, b0, c0, r0buf, sem_g0, sem_s0, sem_i0),
             (si1, di1, sd1, a1, b1, c1, r1buf, sem_g1, sem_s1, sem_i1))

    def copy_idx(di, sdi):
        for o in (0, 16, _CH - 16):
            sl = pl.ds(o, 16)
            sdi[sl] = di[sl]

    def block(j, i, slot, first, last):
        si, di, sdi, a, b, c, res, sem_g, sem_s, sem_i = slots[slot]
        siq, diq, _, aq, bq, cq, _, sem_gq, sem_sq, sem_iq = slots[1 - slot]
        for d in g_descs(j, si, di, a, b, c, sem_g):
            d.wait()
        if first:
            @pl.when(i > 0)
            def _():
                s_desc(r1buf, sd1, sem_s1).wait()
        else:
            s_desc(r0buf, sd0, sem_s0).wait()
        copy_idx(di, sdi)

        def prefetch():
            for d in i_descs(j + 1, siq, diq, sem_iq):
                d.wait()
            for d in g_descs(j + 1, siq, diq, aq, bq, cq, sem_gq):
                d.start()
        if last:
            pl.when(j + 1 < _NCHUNK)(prefetch)
        else:
            prefetch()

        def idx_ahead():
            for d in i_descs(j + 2, si, di, sem_i):
                d.start()
        pl.when(j + 2 < _NCHUNK)(idx_ahead)

        compute(a, b, c, res)
        s_desc(res, sdi, sem_s).start(add=True)

    for d in i_descs(0, si0, di0, sem_i0):
        d.start()
    for d in i_descs(1, si1, di1, sem_i1):
        d.start()
    for d in i_descs(0, si0, di0, sem_i0):
        d.wait()
    for d in g_descs(0, si0, di0, a0, b0, c0, sem_g0):
        d.start()

    def pair_body(i, carry):
        j0 = 2 * i
        block(j0, i, 0, True, False)
        block(j0 + 1, i, 1, False, True)
        return carry

    lax.fori_loop(0, _NCHUNK // 2, pair_body, 0)
    s_desc(r1buf, sd1, sem_s1).wait()

    plsc.subcore_barrier()
    pltpu.sync_copy(acc.at[pl.ds(row0, _ROWS_PER_TILE)],
                    out_hbm.at[cid, pl.ds(row0, _ROWS_PER_TILE)])


_sc_edge = functools.partial(
    pl.kernel,
    out_type=jax.ShapeDtypeStruct((_NC, _N_PAD, C_S), jnp.float32),
    mesh=plsc.VectorSubcoreMesh(core_axis_name="c", subcore_axis_name="s"),
    scratch_types=[
        pltpu.VMEM((_CH,), jnp.int32),
        pltpu.VMEM((_CH,), jnp.int32),
        pltpu.VMEM((_CH,), jnp.int32),
        pltpu.VMEM((_CH,), jnp.int32),
        pltpu.VMEM((_CH,), jnp.int32),
        pltpu.VMEM((_CH,), jnp.int32),
        pltpu.VMEM((_CH, C_S), jnp.float32),
        pltpu.VMEM((_CH, C_S), jnp.float32),
        pltpu.VMEM((_CH, C_S), jnp.float32),
        pltpu.VMEM((_CH, C_S), jnp.float32),
        pltpu.VMEM((_CH, C_S), jnp.float32),
        pltpu.VMEM((_CH, C_S), jnp.float32),
        pltpu.VMEM((_CH, C_S), jnp.float32),
        pltpu.VMEM((_CH, C_S), jnp.float32),
        pltpu.VMEM_SHARED((_N_PAD, C_S), jnp.float32),
        pltpu.SemaphoreType.DMA,
        pltpu.SemaphoreType.DMA,
        pltpu.SemaphoreType.DMA,
        pltpu.SemaphoreType.DMA,
        pltpu.SemaphoreType.DMA,
        pltpu.SemaphoreType.DMA,
    ],
)(_sc_edge_body)


def _sum2_kernel(p_ref, o_ref):
    o_ref[...] = p_ref[0] + p_ref[1]


def _sum_partials(partials):
    rows = 1000
    grid = N_NODES // rows
    return pl.pallas_call(
        _sum2_kernel,
        grid=(grid,),
        in_specs=[pl.BlockSpec((_NC, rows, C_S), lambda i: (0, i, 0))],
        out_specs=pl.BlockSpec((rows, C_S), lambda i: (i, 0)),
        out_shape=jax.ShapeDtypeStruct((N_NODES, C_S), jnp.float32),
    )(partials)


def kernel(node_scalars, rigids_rot, rigids_trans, edge_features, edge_index,
           node_vectors, W_pts, W_comb, W_mlp, b_mlp):
    n = node_scalars.shape[0]
    w1 = W_mlp[:C_S]
    w2 = W_mlp[C_S:2 * C_S]
    w3 = W_mlp[2 * C_S:2 * C_S + C_Z]
    w4 = W_mlp[2 * C_S + C_Z:2 * C_S + C_Z + 3 * C_V]
    w5 = W_mlp[2 * C_S + C_Z + 3 * C_V:2 * C_S + C_Z + 6 * C_V]
    w6 = W_mlp[2 * C_S + C_Z + 6 * C_V:]
    to_cmajor = lambda w: (w.reshape(C_V, 3, C_S).transpose(1, 0, 2)
                           .reshape(3 * C_V, C_S))
    w46 = to_cmajor(w4 + w6)
    w56 = to_cmajor(w5 - w6)
    rot9 = rigids_rot.reshape(n, 9)
    nvt = node_vectors.transpose(0, 2, 1).reshape(n, 3 * C_V)

    vcat, a_tab, b_tab = _node_precompute(
        node_scalars, rot9, rigids_trans, nvt, W_pts, W_comb,
        w1, w2, w46, w56, b_mlp.reshape(1, C_S))
    c_tab = _edge_c(edge_features, w3)

    ei = edge_index.astype(jnp.int32)
    src = ei[0].reshape(_NW, _NCHUNK, _CH)
    dst = ei[1].reshape(_NW, _NCHUNK, _CH)
    zeros = jnp.zeros((_N_PAD, C_S), jnp.float32)
    partials = _sc_edge(a_tab, b_tab, c_tab, src, dst, zeros)
    out_s_s = _sum_partials(partials)
    out_s_v = vcat.reshape(n, 3, C_V).transpose(0, 2, 1)
    return (out_s_s, out_s_v)


# speedup vs baseline: 1.1192x; 1.1192x over previous
import functools

import jax
import jax.numpy as jnp
from jax import lax
from jax.experimental import pallas as pl
from jax.experimental.pallas import tpu as pltpu
from jax.experimental.pallas import tpu_sc as plsc

N_NODES = 10000
N_EDGES = 320000
C_S = 128
C_V = 8
C_Z = 16
NS_PTS = 8
D_POINTS = 10.0

_NC = 2
_NS = 16
_NW = _NC * _NS
_EPW = N_EDGES // _NW
_CH = 40
_NCHUNK = _EPW // _CH
_N_PAD = 10240
_ROWS_PER_TILE = _N_PAD // _NS


def _node_kernel(ns_ref, rot_ref, tr_ref, nvt_ref, wpts_ref, wcomb_ref,
                 w1_ref, w2_ref, w46_ref, w56_ref, b_ref,
                 vcat_ref, a_ref, b_out_ref):
    ns = ns_ref[...]
    rot = rot_ref[...]
    tr = tr_ref[...]
    nvt = nvt_ref[...]
    wpts = wpts_ref[...]
    wcomb = wcomb_ref[...]
    wc_top = wcomb[:C_V, :]
    wc_bot = wcomb[C_V:, :]
    p = jnp.dot(ns, wpts, preferred_element_type=jnp.float32)
    a_acc = jnp.dot(ns, w1_ref[...], preferred_element_type=jnp.float32)
    a_acc = a_acc + b_ref[...]
    b_acc = jnp.dot(ns, w2_ref[...], preferred_element_type=jnp.float32)
    for i in range(3):
        r_i = (rot[:, 3 * i + 0:3 * i + 1] * p[:, 0:NS_PTS]
               + rot[:, 3 * i + 1:3 * i + 2] * p[:, NS_PTS:2 * NS_PTS]
               + rot[:, 3 * i + 2:3 * i + 3] * p[:, 2 * NS_PTS:3 * NS_PTS])
        nv_i = nvt[:, C_V * i:C_V * (i + 1)]
        out_v_i = (jnp.dot(nv_i, wc_top, preferred_element_type=jnp.float32)
                   + jnp.dot(r_i, wc_bot, preferred_element_type=jnp.float32)
                   + tr[:, i:i + 1] * (1.0 / D_POINTS))
        vcat_ref[:, C_V * i:C_V * (i + 1)] = out_v_i
        a_acc = a_acc + jnp.dot(out_v_i, w46_ref[C_V * i:C_V * (i + 1), :],
                                preferred_element_type=jnp.float32)
        b_acc = b_acc + jnp.dot(out_v_i, w56_ref[C_V * i:C_V * (i + 1), :],
                                preferred_element_type=jnp.float32)
    a_ref[...] = a_acc
    b_out_ref[...] = b_acc


def _node_precompute(ns, rot9, trans, nvt, W_pts, W_comb, W1, W2, W46, W56, b):
    rows = 1000
    grid = N_NODES // rows
    full = lambda shape: pl.BlockSpec(shape, lambda i: (0, 0))
    blk = lambda w: pl.BlockSpec((rows, w), lambda i: (i, 0))
    return pl.pallas_call(
        _node_kernel,
        grid=(grid,),
        in_specs=[
            blk(C_S), blk(9), blk(3), blk(3 * C_V),
            full((C_S, 3 * NS_PTS)), full((C_V + NS_PTS, C_V)),
            full((C_S, C_S)), full((C_S, C_S)),
            full((3 * C_V, C_S)), full((3 * C_V, C_S)), full((1, C_S)),
        ],
        out_specs=[blk(3 * C_V), blk(C_S), blk(C_S)],
        out_shape=[
            jax.ShapeDtypeStruct((N_NODES, 3 * C_V), jnp.float32),
            jax.ShapeDtypeStruct((N_NODES, C_S), jnp.float32),
            jax.ShapeDtypeStruct((N_NODES, C_S), jnp.float32),
        ],
    )(ns, rot9, trans, nvt, W_pts, W_comb, W1, W2, W46, W56, b)


def _edgec_kernel(ef_ref, w3_ref, c_ref):
    c_ref[...] = jnp.dot(ef_ref[...], w3_ref[...],
                         preferred_element_type=jnp.float32)


def _edge_c(ef, W3):
    rows = 8000
    grid = N_EDGES // rows
    return pl.pallas_call(
        _edgec_kernel,
        grid=(grid,),
        in_specs=[
            pl.BlockSpec((rows, C_Z), lambda i: (i, 0)),
            pl.BlockSpec((C_Z, C_S), lambda i: (0, 0)),
        ],
        out_specs=pl.BlockSpec((rows, C_S), lambda i: (i, 0)),
        out_shape=jax.ShapeDtypeStruct((N_EDGES, C_S), jnp.float32),
    )(ef, W3)


def _softplus_vec(x):
    t = jnp.exp(jnp.minimum(x, -x))
    p = t * (0.9721037 - 0.2992710 * t)
    return jnp.maximum(x, 0.0) + p


def _sc_edge_body(a_hbm, b_hbm, c_hbm, pk_hbm, z_hbm, out_hbm,
                  pk_all, si0, di0, si1, di1,
                  a0, b0, c0, a1, b1, c1,
                  acc, sem_g0, sem_g1, sem_s0, sem_s1):
    cid = lax.axis_index("c")
    sid = lax.axis_index("s")
    wid = cid * _NS + sid
    cb = wid * _EPW

    row0 = sid * _ROWS_PER_TILE
    pltpu.sync_copy(z_hbm.at[pl.ds(row0, _ROWS_PER_TILE)],
                    acc.at[pl.ds(row0, _ROWS_PER_TILE)])
    pltpu.sync_copy(pk_hbm.at[wid], pk_all)
    plsc.subcore_barrier()

    def unpack(j, si, di):
        offs = list(range(0, _CH - 15, 16))
        if _CH % 16:
            offs.append(_CH - 16)
        for o in offs:
            w = pk_all[pl.ds(j * _CH + o, 16)]
            si[pl.ds(o, 16)] = jnp.bitwise_and(w, 0xFFFF)
            di[pl.ds(o, 16)] = jnp.right_shift(w, 16)

    def g_descs(j, si, di, a, b, c, sem):
        return (pltpu.make_async_copy(a_hbm.at[si], a, sem),
                pltpu.make_async_copy(b_hbm.at[di], b, sem),
                pltpu.make_async_copy(c_hbm.at[pl.ds(cb + j * _CH, _CH)],
                                      c, sem))

    def s_desc(res, di, sem):
        return pltpu.make_async_copy(res, acc.at[di], sem)

    def compute(a, b, c, res):
        def row_body(r):
            for v in range(C_S // 16):
                sl = pl.ds(16 * v, 16)
                res[r, sl] = _softplus_vec(a[r, sl] + b[r, sl] + c[r, sl])
        plsc.parallel_loop(0, _CH, 1, unroll=4)(row_body)

    slots = ((si0, di0, a0, b0, c0, sem_g0, sem_s0),
             (si1, di1, a1, b1, c1, sem_g1, sem_s1))

    def block(j, i, slot, first, last):
        si, di, a, b, c, sem_g, sem_s = slots[slot]
        siq, diq, aq, bq, cq, sem_gq, sem_sq = slots[1 - slot]
        for d in g_descs(j, si, di, a, b, c, sem_g):
            d.wait()
        if first:
            @pl.when(i > 0)
            def _():
                s_desc(a1, di1, sem_s1).wait()
        else:
            s_desc(a0, di0, sem_s0).wait()

        def prefetch():
            unpack(j + 1, siq, diq)
            for d in g_descs(j + 1, siq, diq, aq, bq, cq, sem_gq):
                d.start()
        if last:
            pl.when(j + 1 < _NCHUNK)(prefetch)
        else:
            prefetch()
        compute(a, b, c, a)
        s_desc(a, di, sem_s).start(add=True)

    unpack(0, si0, di0)
    for d in g_descs(0, si0, di0, a0, b0, c0, sem_g0):
        d.start()

    def pair_body(i, carry):
        j0 = 2 * i
        block(j0, i, 0, True, False)
        block(j0 + 1, i, 1, False, True)
        return carry

    lax.fori_loop(0, _NCHUNK // 2, pair_body, 0)
    s_desc(a1, di1, sem_s1).wait()

    plsc.subcore_barrier()
    pltpu.sync_copy(acc.at[pl.ds(row0, _ROWS_PER_TILE)],
                    out_hbm.at[cid, pl.ds(row0, _ROWS_PER_TILE)])


_sc_edge = functools.partial(
    pl.kernel,
    out_type=jax.ShapeDtypeStruct((_NC, _N_PAD, C_S), jnp.float32),
    mesh=plsc.VectorSubcoreMesh(core_axis_name="c", subcore_axis_name="s"),
    scratch_types=[
        pltpu.VMEM((_EPW,), jnp.int32),
        pltpu.VMEM((_CH,), jnp.int32),
        pltpu.VMEM((_CH,), jnp.int32),
        pltpu.VMEM((_CH,), jnp.int32),
        pltpu.VMEM((_CH,), jnp.int32),
        pltpu.VMEM((_CH, C_S), jnp.float32),
        pltpu.VMEM((_CH, C_S), jnp.float32),
        pltpu.VMEM((_CH, C_S), jnp.float32),
        pltpu.VMEM((_CH, C_S), jnp.float32),
        pltpu.VMEM((_CH, C_S), jnp.float32),
        pltpu.VMEM((_CH, C_S), jnp.float32),
        pltpu.VMEM_SHARED((_N_PAD, C_S), jnp.float32),
        pltpu.SemaphoreType.DMA,
        pltpu.SemaphoreType.DMA,
        pltpu.SemaphoreType.DMA,
        pltpu.SemaphoreType.DMA,
    ],
)(_sc_edge_body)


def _sum2_kernel(p_ref, o_ref):
    o_ref[...] = p_ref[0] + p_ref[1]


def _sum_partials(partials):
    rows = 1000
    grid = N_NODES // rows
    return pl.pallas_call(
        _sum2_kernel,
        grid=(grid,),
        in_specs=[pl.BlockSpec((_NC, rows, C_S), lambda i: (0, i, 0))],
        out_specs=pl.BlockSpec((rows, C_S), lambda i: (i, 0)),
        out_shape=jax.ShapeDtypeStruct((N_NODES, C_S), jnp.float32),
    )(partials)


def kernel(node_scalars, rigids_rot, rigids_trans, edge_features, edge_index,
           node_vectors, W_pts, W_comb, W_mlp, b_mlp):
    n = node_scalars.shape[0]
    w1 = W_mlp[:C_S]
    w2 = W_mlp[C_S:2 * C_S]
    w3 = W_mlp[2 * C_S:2 * C_S + C_Z]
    w4 = W_mlp[2 * C_S + C_Z:2 * C_S + C_Z + 3 * C_V]
    w5 = W_mlp[2 * C_S + C_Z + 3 * C_V:2 * C_S + C_Z + 6 * C_V]
    w6 = W_mlp[2 * C_S + C_Z + 6 * C_V:]
    to_cmajor = lambda w: (w.reshape(C_V, 3, C_S).transpose(1, 0, 2)
                           .reshape(3 * C_V, C_S))
    w46 = to_cmajor(w4 + w6)
    w56 = to_cmajor(w5 - w6)
    rot9 = rigids_rot.reshape(n, 9)
    nvt = node_vectors.transpose(0, 2, 1).reshape(n, 3 * C_V)

    vcat, a_tab, b_tab = _node_precompute(
        node_scalars, rot9, rigids_trans, nvt, W_pts, W_comb,
        w1, w2, w46, w56, b_mlp.reshape(1, C_S))
    c_tab = _edge_c(edge_features, w3)

    ei = edge_index.astype(jnp.int32)
    packed = jnp.bitwise_or(ei[0], jnp.left_shift(ei[1], 16))
    packed = packed.reshape(_NW, _EPW)
    zeros = jnp.zeros((_N_PAD, C_S), jnp.float32)
    partials = _sc_edge(a_tab, b_tab, c_tab, packed, zeros)
    out_s_s = _sum_partials(partials)
    out_s_v = vcat.reshape(n, 3, C_V).transpose(0, 2, 1)
    return (out_s_s, out_s_v)
